# baseline (speedup 1.0000x reference)
import functools

import jax
import jax.numpy as jnp
from jax import lax
from jax.experimental import pallas as pl
from jax.experimental.pallas import tpu as pltpu
from jax.experimental.pallas import tpu_sc as plsc

_NC = 2
_NS = 16
_NW = _NC * _NS
_LANES = 16
_CHUNK = 128


def kernel(input_ids, position_ids, subword_table, m1_table, m2_table):
    B, L = input_ids.shape
    D = subword_table.shape[1]
    N = B * L
    per_w = N // _NW
    n_chunks = per_w // _CHUNK
    assert N % (_NW * _CHUNK) == 0 and D % _LANES == 0

    pm1 = jnp.tile(jnp.mod(position_ids, 47).astype(jnp.int32), 2)
    pm2 = jnp.tile(jnp.mod(position_ids, 11).astype(jnp.int32), 2)
    idx = input_ids.astype(jnp.int32).reshape(_NW, n_chunks, _CHUNK)

    mesh = plsc.VectorSubcoreMesh(core_axis_name="c", subcore_axis_name="s")

    @functools.partial(
        pl.kernel,
        out_type=jax.ShapeDtypeStruct((N, D), jnp.float32),
        mesh=mesh,
        compiler_params=pltpu.CompilerParams(use_tc_tiling_on_sc=False),
        scratch_types=[
            pltpu.VMEM((n_chunks, _CHUNK), jnp.int32),
            pltpu.VMEM((_CHUNK, D), jnp.float32),
            pltpu.VMEM((2 * L, D), jnp.float32),
            pltpu.VMEM((2 * L, D), jnp.float32),
            pltpu.VMEM((2 * L,), jnp.int32),
            pltpu.VMEM((2 * L,), jnp.int32),
            pltpu.SemaphoreType.DMA,
        ],
    )
    def k(idx_hbm, pm1_hbm, pm2_hbm, table_hbm, m1_hbm, m2_hbm, out_hbm,
          idx_v, rows_v, pos_v, tmp_v, pm1_v, pm2_v, sem):
        wid = lax.axis_index("s") * _NC + lax.axis_index("c")
        base = wid * per_w

        pltpu.sync_copy(pm1_hbm, pm1_v)
        pltpu.sync_copy(pm2_hbm, pm2_v)
        pltpu.async_copy(m1_hbm.at[pm1_v], tmp_v, sem).wait()
        pltpu.async_copy(m2_hbm.at[pm2_v], pos_v, sem).wait()

        @pl.loop(0, 2 * L)
        def _(r):
            for c in range(D // _LANES):
                sl = pl.ds(c * _LANES, _LANES)
                pos_v[r, sl] = pos_v[r, sl] + tmp_v[r, sl]

        pltpu.sync_copy(idx_hbm.at[wid], idx_v)

        @pl.loop(0, n_chunks)
        def _(g):
            pltpu.async_copy(table_hbm.at[idx_v.at[g]], rows_v, sem).wait()
            off = lax.rem(g * _CHUNK, L)

            @pl.loop(0, _CHUNK)
            def _(j):
                for c in range(D // _LANES):
                    sl = pl.ds(c * _LANES, _LANES)
                    rows_v[j, sl] = rows_v[j, sl] + pos_v[off + j, sl]

            pltpu.sync_copy(rows_v, out_hbm.at[pl.ds(base + g * _CHUNK, _CHUNK)])

    out = k(idx, pm1, pm2, subword_table, m1_table, m2_table)
    return out.reshape(B, L, D)


# speedup vs baseline: 1.1774x; 1.1774x over previous
import functools

import jax
import jax.numpy as jnp
from jax import lax
from jax.experimental import pallas as pl
from jax.experimental.pallas import tpu as pltpu
from jax.experimental.pallas import tpu_sc as plsc

_NC = 2
_NS = 16
_NW = _NC * _NS
_LANES = 16
_CHUNK = 128
_NBUF = 4


def kernel(input_ids, position_ids, subword_table, m1_table, m2_table):
    B, L = input_ids.shape
    D = subword_table.shape[1]
    N = B * L
    per_w = N // _NW
    n_chunks = per_w // _CHUNK
    assert N % (_NW * _CHUNK) == 0 and D % _LANES == 0
    assert n_chunks % _NBUF == 0 and n_chunks >= 3 * _NBUF

    pm1 = jnp.tile(jnp.mod(position_ids, 47).astype(jnp.int32), 2)
    pm2 = jnp.tile(jnp.mod(position_ids, 11).astype(jnp.int32), 2)
    idx = input_ids.astype(jnp.int32).reshape(_NW, n_chunks, _CHUNK)

    mesh = plsc.VectorSubcoreMesh(core_axis_name="c", subcore_axis_name="s")

    row_buf = pltpu.VMEM((_CHUNK, D), jnp.float32)

    @functools.partial(
        pl.kernel,
        out_type=jax.ShapeDtypeStruct((N, D), jnp.float32),
        mesh=mesh,
        compiler_params=pltpu.CompilerParams(use_tc_tiling_on_sc=False),
        scratch_types=[
            pltpu.VMEM((n_chunks, _CHUNK), jnp.int32),
            [row_buf] * _NBUF,
            pltpu.VMEM((2 * L, D), jnp.float32),
            pltpu.VMEM((2 * L, D), jnp.float32),
            pltpu.VMEM((2 * L,), jnp.int32),
            pltpu.VMEM((2 * L,), jnp.int32),
            [pltpu.SemaphoreType.DMA] * _NBUF,
            [pltpu.SemaphoreType.DMA] * _NBUF,
        ],
    )
    def k(idx_hbm, pm1_hbm, pm2_hbm, table_hbm, m1_hbm, m2_hbm,
          out_hbm, idx_v, rows, pos_v, tmp_v, pm1_v, pm2_v,
          gsem, osem):
        wid = lax.axis_index("s") * _NC + lax.axis_index("c")
        base = wid * per_w

        pltpu.sync_copy(pm1_hbm, pm1_v)
        pltpu.sync_copy(pm2_hbm, pm2_v)
        pltpu.async_copy(m1_hbm.at[pm1_v], tmp_v, gsem[0]).wait()
        pltpu.async_copy(m2_hbm.at[pm2_v], pos_v, gsem[0]).wait()

        @pl.loop(0, 2 * L)
        def _(r):
            for c in range(D // _LANES):
                sl = pl.ds(c * _LANES, _LANES)
                pos_v[r, sl] = pos_v[r, sl] + tmp_v[r, sl]

        pltpu.sync_copy(idx_hbm.at[wid], idx_v)

        def issue_gather(t, b):
            pltpu.async_copy(table_hbm.at[idx_v.at[t]], rows[b], gsem[b])

        def wait_gather(b):
            pltpu.make_async_copy(
                table_hbm.at[idx_v.at[0]], rows[b], gsem[b]).wait()

        def add_pos(t, b):
            off = lax.rem(t * _CHUNK, L)

            @pl.loop(0, _CHUNK, step=4)
            def _(j):
                for jj in range(4):
                    for c in range(D // _LANES):
                        sl = pl.ds(c * _LANES, _LANES)
                        rows[b][j + jj, sl] = (
                            rows[b][j + jj, sl] + pos_v[off + j + jj, sl])

        def issue_out(t, b):
            pltpu.async_copy(
                rows[b], out_hbm.at[pl.ds(base + t * _CHUNK, _CHUNK)], osem[b])

        def wait_out(b):
            pltpu.make_async_copy(
                rows[b], out_hbm.at[pl.ds(base, _CHUNK)], osem[b]).wait()

        issue_gather(0, 0)
        issue_gather(1, 1)
        for t in range(4):
            b = t % _NBUF
            wait_gather(b)
            add_pos(t, b)
            issue_out(t, b)
            b2 = (t + 2) % _NBUF
            if t >= 2:
                wait_out(b2)
            issue_gather(t + 2, b2)

        @pl.loop(1, n_chunks // _NBUF - 1)
        def _(i):
            for b in range(_NBUF):
                t = _NBUF * i + b
                wait_gather(b)
                add_pos(t, b)
                issue_out(t, b)
                b2 = (b + 2) % _NBUF
                wait_out(b2)
                issue_gather(t + 2, b2)

        for t in range(n_chunks - 4, n_chunks):
            b = t % _NBUF
            wait_gather(b)
            add_pos(t, b)
            issue_out(t, b)
            b2 = (b + 2) % _NBUF
            if t + 2 < n_chunks:
                wait_out(b2)
                issue_gather(t + 2, b2)

        for b in range(_NBUF):
            wait_out(b)

    out = k(idx, pm1, pm2, subword_table, m1_table, m2_table)
    return out.reshape(B, L, D)


# speedup vs baseline: 1.2683x; 1.0773x over previous
import functools

import jax
import jax.numpy as jnp
from jax import lax
from jax.experimental import pallas as pl
from jax.experimental.pallas import tpu as pltpu
from jax.experimental.pallas import tpu_sc as plsc

_NC = 2
_NS = 16
_NW = _NC * _NS
_LANES = 16
_CHUNK = 128
_NBUF = 8
_LOOK = 4


def kernel(input_ids, position_ids, subword_table, m1_table, m2_table):
    B, L = input_ids.shape
    D = subword_table.shape[1]
    N = B * L
    per_w = N // _NW
    n_chunks = per_w // _CHUNK
    assert N % (_NW * _CHUNK) == 0 and D % _LANES == 0
    assert n_chunks % _NBUF == 0 and n_chunks >= 3 * _NBUF
    assert _NBUF * _CHUNK >= 2 * L

    pm1 = jnp.tile(jnp.mod(position_ids, 47).astype(jnp.int32), 2)
    pm2 = jnp.tile(jnp.mod(position_ids, 11).astype(jnp.int32), 2)
    idx = input_ids.astype(jnp.int32).reshape(_NW, n_chunks, _CHUNK)

    mesh = plsc.VectorSubcoreMesh(core_axis_name="c", subcore_axis_name="s")

    @functools.partial(
        pl.kernel,
        out_type=jax.ShapeDtypeStruct((N, D), jnp.float32),
        mesh=mesh,
        compiler_params=pltpu.CompilerParams(use_tc_tiling_on_sc=False),
        scratch_types=[
            pltpu.VMEM((n_chunks, _CHUNK), jnp.int32),
            pltpu.VMEM((_NBUF * _CHUNK, D), jnp.float32),
            pltpu.VMEM((2 * L, D), jnp.float32),
            pltpu.VMEM((2 * L,), jnp.int32),
            pltpu.VMEM((2 * L,), jnp.int32),
            [pltpu.SemaphoreType.DMA] * _NBUF,
            [pltpu.SemaphoreType.DMA] * _NBUF,
        ],
    )
    def k(idx_hbm, pm1_hbm, pm2_hbm, table_hbm, m1_hbm, m2_hbm,
          out_hbm, idx_v, rows, pos_v, pm1_v, pm2_v, gsem, osem):
        wid = lax.axis_index("s") * _NC + lax.axis_index("c")
        base = wid * per_w

        def rows_sl(b):
            return rows.at[pl.ds(b * _CHUNK, _CHUNK), :]

        pltpu.sync_copy(pm1_hbm, pm1_v)
        pltpu.sync_copy(pm2_hbm, pm2_v)
        m1rows = rows.at[pl.ds(0, 2 * L), :]
        pltpu.async_copy(m1_hbm.at[pm1_v], m1rows, gsem[0]).wait()
        pltpu.async_copy(m2_hbm.at[pm2_v], pos_v, gsem[0]).wait()

        @pl.loop(0, 2 * L)
        def _(r):
            for c in range(D // _LANES):
                sl = pl.ds(c * _LANES, _LANES)
                plsc.addupdate(pos_v.at[r, sl], rows[r, sl])

        pltpu.sync_copy(idx_hbm.at[wid], idx_v)

        def issue_gather(t, b):
            pltpu.async_copy(table_hbm.at[idx_v.at[t]], rows_sl(b), gsem[b])

        def wait_gather(b):
            pltpu.make_async_copy(
                table_hbm.at[idx_v.at[0]], rows_sl(b), gsem[b]).wait()

        def add_pos(t, b):
            off = lax.rem(t * _CHUNK, L)

            @pl.loop(0, _CHUNK, step=8)
            def _(j):
                for jj in range(8):
                    for c in range(D // _LANES):
                        sl = pl.ds(c * _LANES, _LANES)
                        plsc.addupdate(rows.at[b * _CHUNK + j + jj, sl],
                                       pos_v[off + j + jj, sl])

        def issue_out(t, b):
            pltpu.async_copy(rows_sl(b),
                             out_hbm.at[pl.ds(base + t * _CHUNK, _CHUNK)],
                             osem[b])

        def wait_out(b):
            pltpu.make_async_copy(
                rows_sl(b), out_hbm.at[pl.ds(base, _CHUNK)], osem[b]).wait()

        for b in range(_LOOK):
            issue_gather(b, b)
        for t in range(_NBUF):
            b = t % _NBUF
            wait_gather(b)
            add_pos(t, b)
            issue_out(t, b)
            b2 = (t + _LOOK) % _NBUF
            if t + _LOOK >= _NBUF:
                wait_out(b2)
            issue_gather(t + _LOOK, b2)

        @pl.loop(1, n_chunks // _NBUF - 1)
        def _(i):
            for b in range(_NBUF):
                t = _NBUF * i + b
                wait_gather(b)
                add_pos(t, b)
                issue_out(t, b)
                b2 = (b + _LOOK) % _NBUF
                wait_out(b2)
                issue_gather(t + _LOOK, b2)

        for t in range(n_chunks - _NBUF, n_chunks):
            b = t % _NBUF
            wait_gather(b)
            add_pos(t, b)
            issue_out(t, b)
            if t + _LOOK < n_chunks:
                b2 = (b + _LOOK) % _NBUF
                wait_out(b2)
                issue_gather(t + _LOOK, b2)

        for b in range(_NBUF):
            wait_out(b)

    out = k(idx, pm1, pm2, subword_table, m1_table, m2_table)
    return out.reshape(B, L, D)


# speedup vs baseline: 1.5307x; 1.2068x over previous
import functools

import jax
import jax.numpy as jnp
from jax import lax
from jax.experimental import pallas as pl
from jax.experimental.pallas import tpu as pltpu
from jax.experimental.pallas import tpu_sc as plsc

_NC = 2
_NS = 16
_NW = _NC * _NS
_LANES = 16
_NBUF = 4
_LOOK = 2


def kernel(input_ids, position_ids, subword_table, m1_table, m2_table):
    B, L = input_ids.shape
    D = subword_table.shape[1]
    n_chunks = B // _NW
    assert B % _NW == 0 and L % 8 == 0 and D % _LANES == 0
    assert n_chunks % _NBUF == 0 and n_chunks >= 3 * _NBUF

    pm1 = jnp.mod(position_ids, 47).astype(jnp.int32)
    pm2 = jnp.mod(position_ids, 11).astype(jnp.int32)
    idx = input_ids.astype(jnp.int32).reshape(_NW, n_chunks, L)

    mesh = plsc.VectorSubcoreMesh(core_axis_name="c", subcore_axis_name="s")

    @functools.partial(
        pl.kernel,
        out_type=jax.ShapeDtypeStruct((B, L, D), jnp.float32),
        mesh=mesh,
        compiler_params=pltpu.CompilerParams(use_tc_tiling_on_sc=False),
        scratch_types=[
            pltpu.VMEM((n_chunks, L), jnp.int32),
            pltpu.VMEM((_NBUF * L, D), jnp.float32),
            pltpu.VMEM((L, D), jnp.float32),
            pltpu.VMEM((L, D), jnp.float32),
            pltpu.VMEM((L,), jnp.int32),
            pltpu.VMEM((L,), jnp.int32),
            [pltpu.SemaphoreType.DMA] * _NBUF,
            [pltpu.SemaphoreType.DMA] * _NBUF,
        ],
    )
    def k(idx_hbm, pm1_hbm, pm2_hbm, table_hbm, m1_hbm, m2_hbm,
          out_hbm, idx_v, rows, pos_v, tmp_v, pm1_v, pm2_v, gsem, osem):
        wid = lax.axis_index("s") * _NC + lax.axis_index("c")
        b0 = wid * n_chunks

        def rows_sl(b):
            return rows.at[pl.ds(b * L, L), :]

        pltpu.sync_copy(pm1_hbm, pm1_v)
        pltpu.sync_copy(pm2_hbm, pm2_v)
        pltpu.async_copy(m1_hbm.at[pm1_v], tmp_v, gsem[0]).wait()
        pltpu.async_copy(m2_hbm.at[pm2_v], pos_v, gsem[0]).wait()

        @pl.loop(0, L)
        def _(r):
            for c in range(D // _LANES):
                sl = pl.ds(c * _LANES, _LANES)
                plsc.addupdate(pos_v.at[r, sl], tmp_v[r, sl])

        pltpu.sync_copy(idx_hbm.at[wid], idx_v)

        def issue_gather(t, b):
            pltpu.async_copy(table_hbm.at[idx_v.at[t]], rows_sl(b), gsem[b])

        def wait_gather(b):
            pltpu.make_async_copy(
                table_hbm.at[idx_v.at[0]], rows_sl(b), gsem[b]).wait()

        def add_pos(t, b):
            @pl.loop(0, L, step=8)
            def _(j):
                for jj in range(8):
                    for c in range(D // _LANES):
                        sl = pl.ds(c * _LANES, _LANES)
                        plsc.addupdate(rows.at[b * L + j + jj, sl],
                                       pos_v[j + jj, sl])

        def issue_out(t, b):
            pltpu.async_copy(rows_sl(b), out_hbm.at[b0 + t], osem[b])

        def wait_out(b):
            pltpu.make_async_copy(rows_sl(b), out_hbm.at[b0], osem[b]).wait()

        for b in range(_LOOK):
            issue_gather(b, b)
        for t in range(_NBUF):
            b = t % _NBUF
            wait_gather(b)
            add_pos(t, b)
            issue_out(t, b)
            b2 = (t + _LOOK) % _NBUF
            if t + _LOOK >= _NBUF:
                wait_out(b2)
            issue_gather(t + _LOOK, b2)

        @pl.loop(1, n_chunks // _NBUF - 1)
        def _(i):
            for b in range(_NBUF):
                t = _NBUF * i + b
                wait_gather(b)
                add_pos(t, b)
                issue_out(t, b)
                b2 = (b + _LOOK) % _NBUF
                wait_out(b2)
                issue_gather(t + _LOOK, b2)

        for t in range(n_chunks - _NBUF, n_chunks):
            b = t % _NBUF
            wait_gather(b)
            add_pos(t, b)
            issue_out(t, b)
            if t + _LOOK < n_chunks:
                b2 = (b + _LOOK) % _NBUF
                wait_out(b2)
                issue_gather(t + _LOOK, b2)

        for b in range(_NBUF):
            wait_out(b)

    return k(idx, pm1, pm2, subword_table, m1_table, m2_table)
